# baseline (speedup 1.0000x reference)
import functools

import jax
import jax.numpy as jnp
from jax import lax
from jax.experimental import pallas as pl
from jax.experimental.pallas import tpu as pltpu
from jax.experimental.pallas import tpu_sc as plsc


def _sc_gather(ids_flat, table):
    N = ids_flat.shape[0]
    V, D = table.shape
    info = plsc.get_sparse_core_info()
    nw = info.num_cores * info.num_subcores
    rows_per_w = N // nw
    ch = 32
    n_ch = rows_per_w // ch
    mesh = plsc.VectorSubcoreMesh(core_axis_name="c", subcore_axis_name="s")

    nb = min(3, n_ch)

    @functools.partial(
        pl.kernel,
        mesh=mesh,
        out_type=jax.ShapeDtypeStruct((N, D), jnp.float32),
        scratch_types=[
            pltpu.VMEM((rows_per_w,), jnp.int32),
        ]
        + [pltpu.VMEM((ch, D), jnp.float32) for _ in range(nb)]
        + [pltpu.SemaphoreType.DMA for _ in range(2 * nb)],
    )
    def gather_kernel(ids_hbm, table_hbm, out_hbm, idx_v, *bufs_sems):
        rows = bufs_sems[:nb]
        gsem = bufs_sems[nb:2 * nb]
        ssem = bufs_sems[2 * nb:]
        wid = lax.axis_index("s") * info.num_cores + lax.axis_index("c")
        base = wid * rows_per_w

        def idx_at(t):
            return idx_v.at[pl.ds(t * ch, ch)]

        pltpu.sync_copy(ids_hbm.at[pl.ds(base, rows_per_w)], idx_v)
        pltpu.make_async_copy(table_hbm.at[idx_at(0)], rows[0], gsem[0]).start()
        for t in range(n_ch):
            j = t % nb
            if t + 1 < n_ch:
                jn = (t + 1) % nb
                if t + 1 >= nb:
                    pltpu.make_async_copy(
                        rows[jn], out_hbm.at[pl.ds(0, ch)], ssem[jn]).wait()
                pltpu.make_async_copy(
                    table_hbm.at[idx_at(t + 1)], rows[jn], gsem[jn]).start()
            pltpu.make_async_copy(
                table_hbm.at[idx_at(t)], rows[j], gsem[j]).wait()
            pltpu.make_async_copy(
                rows[j], out_hbm.at[pl.ds(base + t * ch, ch)], ssem[j]).start()
        for t in range(max(0, n_ch - nb), n_ch):
            j = t % nb
            pltpu.make_async_copy(
                rows[j], out_hbm.at[pl.ds(0, ch)], ssem[j]).wait()

    return gather_kernel(ids_flat, table)


def _ln_body(g_ref, p_ref, tw_ref, tb_ref, pw_ref, pb_ref, o_ref):
    x = g_ref[...]
    mu = jnp.mean(x, axis=-1, keepdims=True)
    var = jnp.mean((x - mu) ** 2, axis=-1, keepdims=True)
    tok = (x - mu) * lax.rsqrt(var + 1e-5) * tw_ref[...] + tb_ref[...]
    p = p_ref[...]
    pmu = jnp.mean(p, axis=-1, keepdims=True)
    pvar = jnp.mean((p - pmu) ** 2, axis=-1, keepdims=True)
    pos = (p - pmu) * lax.rsqrt(pvar + 1e-5) * pw_ref[...] + pb_ref[...]
    o_ref[...] = tok + pos


def _tc_ln_chunk(g, pos_table, tw, tb, pw, pb, buf, k, N, B, S, s_chunk):
    D = g.shape[1]
    blk = 512
    sb = s_chunk // blk
    vec = lambda: pl.BlockSpec((1, D), lambda s, b: (0, 0))
    in_specs = [
        pl.BlockSpec((blk, D), lambda s, b: (b * sb + s, 0)),
        pl.BlockSpec((blk, D), lambda s, b: (k * sb + s, 0)),
        vec(), vec(), vec(), vec(),
    ]
    args = [g, pos_table, tw.reshape(1, D), tb.reshape(1, D),
            pw.reshape(1, D), pb.reshape(1, D)]
    kwargs = {}
    if buf is not None:
        in_specs.append(pl.BlockSpec(memory_space=pl.ANY))
        args.append(buf)
        kwargs["input_output_aliases"] = {6: 0}
        body = lambda g_, p_, a_, b_, c_, d_, _buf, o_: _ln_body(
            g_, p_, a_, b_, c_, d_, o_)
    else:
        body = _ln_body
    return pl.pallas_call(
        body,
        grid=(sb, B),
        in_specs=in_specs,
        out_specs=pl.BlockSpec(
            (blk, D), lambda s, b: (b * (S // blk) + k * sb + s, 0)),
        out_shape=jax.ShapeDtypeStruct((N, D), jnp.float32),
        **kwargs,
    )(*args)


def kernel(input_ids, token_table, pos_table, tok_ln_w, tok_ln_b, pos_ln_w, pos_ln_b):
    B, S = input_ids.shape
    V, D = token_table.shape
    n_chunks = 8
    s_chunk = S // n_chunks
    ids32 = input_ids.astype(jnp.int32)
    gs = [
        _sc_gather(ids32[:, k * s_chunk:(k + 1) * s_chunk].reshape(-1),
                   token_table)
        for k in range(n_chunks)
    ]
    buf = None
    for k in range(n_chunks):
        buf = _tc_ln_chunk(gs[k], pos_table, tok_ln_w, tok_ln_b, pos_ln_w,
                           pos_ln_b, buf, k, B * S, B, S, s_chunk)
    return buf.reshape(B, S, D)


# speedup vs baseline: 1.0311x; 1.0311x over previous
import functools

import jax
import jax.numpy as jnp
from jax import lax
from jax.experimental import pallas as pl
from jax.experimental.pallas import tpu as pltpu
from jax.experimental.pallas import tpu_sc as plsc


def _sc_gather(ids_flat, table):
    N = ids_flat.shape[0]
    V, D = table.shape
    info = plsc.get_sparse_core_info()
    nw = info.num_cores * info.num_subcores
    rows_per_w = N // nw
    ch = 32
    n_ch = rows_per_w // ch
    mesh = plsc.VectorSubcoreMesh(core_axis_name="c", subcore_axis_name="s")

    nb = min(3, n_ch)

    @functools.partial(
        pl.kernel,
        mesh=mesh,
        out_type=jax.ShapeDtypeStruct((N, D), jnp.float32),
        scratch_types=[
            pltpu.VMEM((rows_per_w,), jnp.int32),
        ]
        + [pltpu.VMEM((ch, D), jnp.float32) for _ in range(nb)]
        + [pltpu.SemaphoreType.DMA for _ in range(2 * nb)],
    )
    def gather_kernel(ids_hbm, table_hbm, out_hbm, idx_v, *bufs_sems):
        rows = bufs_sems[:nb]
        gsem = bufs_sems[nb:2 * nb]
        ssem = bufs_sems[2 * nb:]
        wid = lax.axis_index("s") * info.num_cores + lax.axis_index("c")
        base = wid * rows_per_w

        def idx_at(t):
            return idx_v.at[pl.ds(t * ch, ch)]

        pltpu.sync_copy(ids_hbm.at[pl.ds(base, rows_per_w)], idx_v)
        pltpu.make_async_copy(table_hbm.at[idx_at(0)], rows[0], gsem[0]).start()
        for t in range(n_ch):
            j = t % nb
            if t + 1 < n_ch:
                jn = (t + 1) % nb
                if t + 1 >= nb:
                    pltpu.make_async_copy(
                        rows[jn], out_hbm.at[pl.ds(0, ch)], ssem[jn]).wait()
                pltpu.make_async_copy(
                    table_hbm.at[idx_at(t + 1)], rows[jn], gsem[jn]).start()
            pltpu.make_async_copy(
                table_hbm.at[idx_at(t)], rows[j], gsem[j]).wait()
            pltpu.make_async_copy(
                rows[j], out_hbm.at[pl.ds(base + t * ch, ch)], ssem[j]).start()
        for t in range(max(0, n_ch - nb), n_ch):
            j = t % nb
            pltpu.make_async_copy(
                rows[j], out_hbm.at[pl.ds(0, ch)], ssem[j]).wait()

    return gather_kernel(ids_flat, table)


def _ln_body(g_ref, p_ref, tw_ref, tb_ref, pw_ref, pb_ref, o_ref):
    x = g_ref[...]
    mu = jnp.mean(x, axis=-1, keepdims=True)
    var = jnp.mean((x - mu) ** 2, axis=-1, keepdims=True)
    tok = (x - mu) * lax.rsqrt(var + 1e-5) * tw_ref[...] + tb_ref[...]
    p = p_ref[...]
    pmu = jnp.mean(p, axis=-1, keepdims=True)
    pvar = jnp.mean((p - pmu) ** 2, axis=-1, keepdims=True)
    pos = (p - pmu) * lax.rsqrt(pvar + 1e-5) * pw_ref[...] + pb_ref[...]
    o_ref[...] = tok + pos


def _tc_ln_chunk(g, pos_table, tw, tb, pw, pb, buf, k_blk, N, B, S, s_chunk):
    D = g.shape[1]
    blk = 512
    sb = s_chunk // blk
    vec = lambda: pl.BlockSpec((1, D), lambda s, b: (0, 0))
    in_specs = [
        pl.BlockSpec((blk, D), lambda s, b: (b * sb + s, 0)),
        pl.BlockSpec((blk, D), lambda s, b: (k_blk + s, 0)),
        vec(), vec(), vec(), vec(),
    ]
    args = [g, pos_table, tw.reshape(1, D), tb.reshape(1, D),
            pw.reshape(1, D), pb.reshape(1, D)]
    kwargs = {}
    if buf is not None:
        in_specs.append(pl.BlockSpec(memory_space=pl.ANY))
        args.append(buf)
        kwargs["input_output_aliases"] = {6: 0}
        body = lambda g_, p_, a_, b_, c_, d_, _buf, o_: _ln_body(
            g_, p_, a_, b_, c_, d_, o_)
    else:
        body = _ln_body
    return pl.pallas_call(
        body,
        grid=(sb, B),
        in_specs=in_specs,
        out_specs=pl.BlockSpec(
            (blk, D), lambda s, b: (b * (S // blk) + k_blk + s, 0)),
        out_shape=jax.ShapeDtypeStruct((N, D), jnp.float32),
        **kwargs,
    )(*args)


def kernel(input_ids, token_table, pos_table, tok_ln_w, tok_ln_b, pos_ln_w, pos_ln_b):
    B, S = input_ids.shape
    V, D = token_table.shape
    sizes = (512, 1536, 1536, 512)
    offs = (0, 512, 2048, 3584)
    ids32 = input_ids.astype(jnp.int32)
    gs = [
        _sc_gather(ids32[:, o:o + sz].reshape(-1), token_table)
        for o, sz in zip(offs, sizes)
    ]
    buf = None
    for g, o, sz in zip(gs, offs, sizes):
        buf = _tc_ln_chunk(g, pos_table, tok_ln_w, tok_ln_b, pos_ln_w,
                           pos_ln_b, buf, o // 512, B * S, B, S, sz)
    return buf.reshape(B, S, D)


# speedup vs baseline: 1.0847x; 1.0520x over previous
import functools

import jax
import jax.numpy as jnp
from jax import lax
from jax.experimental import pallas as pl
from jax.experimental.pallas import tpu as pltpu
from jax.experimental.pallas import tpu_sc as plsc


def _sc_gather(ids_flat, table):
    N = ids_flat.shape[0]
    V, D = table.shape
    info = plsc.get_sparse_core_info()
    nw = info.num_cores * info.num_subcores
    rows_per_w = N // nw
    ch = 32
    n_ch = rows_per_w // ch
    mesh = plsc.VectorSubcoreMesh(core_axis_name="c", subcore_axis_name="s")

    nb = min(3, n_ch)

    @functools.partial(
        pl.kernel,
        mesh=mesh,
        out_type=jax.ShapeDtypeStruct((N, D), jnp.float32),
        scratch_types=[
            pltpu.VMEM((rows_per_w,), jnp.int32),
        ]
        + [pltpu.VMEM((ch, D), jnp.float32) for _ in range(nb)]
        + [pltpu.SemaphoreType.DMA for _ in range(2 * nb)],
    )
    def gather_kernel(ids_hbm, table_hbm, out_hbm, idx_v, *bufs_sems):
        rows = bufs_sems[:nb]
        gsem = bufs_sems[nb:2 * nb]
        ssem = bufs_sems[2 * nb:]
        wid = lax.axis_index("s") * info.num_cores + lax.axis_index("c")
        base = wid * rows_per_w

        def idx_at(t):
            return idx_v.at[pl.ds(t * ch, ch)]

        pltpu.sync_copy(ids_hbm.at[pl.ds(base, rows_per_w)], idx_v)
        pltpu.make_async_copy(table_hbm.at[idx_at(0)], rows[0], gsem[0]).start()
        for t in range(n_ch):
            j = t % nb
            if t + 1 < n_ch:
                jn = (t + 1) % nb
                if t + 1 >= nb:
                    pltpu.make_async_copy(
                        rows[jn], out_hbm.at[pl.ds(0, ch)], ssem[jn]).wait()
                pltpu.make_async_copy(
                    table_hbm.at[idx_at(t + 1)], rows[jn], gsem[jn]).start()
            pltpu.make_async_copy(
                table_hbm.at[idx_at(t)], rows[j], gsem[j]).wait()
            pltpu.make_async_copy(
                rows[j], out_hbm.at[pl.ds(base + t * ch, ch)], ssem[j]).start()
        for t in range(max(0, n_ch - nb), n_ch):
            j = t % nb
            pltpu.make_async_copy(
                rows[j], out_hbm.at[pl.ds(0, ch)], ssem[j]).wait()

    return gather_kernel(ids_flat, table)


def _ln_body(g_ref, p_ref, tw_ref, tb_ref, pw_ref, pb_ref, o_ref):
    x = g_ref[...]
    mu = jnp.mean(x, axis=-1, keepdims=True)
    var = jnp.mean((x - mu) ** 2, axis=-1, keepdims=True)
    tok = (x - mu) * lax.rsqrt(var + 1e-5) * tw_ref[...] + tb_ref[...]
    p = p_ref[...]
    pmu = jnp.mean(p, axis=-1, keepdims=True)
    pvar = jnp.mean((p - pmu) ** 2, axis=-1, keepdims=True)
    pos = (p - pmu) * lax.rsqrt(pvar + 1e-5) * pw_ref[...] + pb_ref[...]
    o_ref[...] = tok + pos


def _tc_ln_chunk(g, pos_table, tw, tb, pw, pb, buf, k_blk, N, B, S, s_chunk):
    D = g.shape[1]
    blk = 1024
    sb = s_chunk // blk
    vec = lambda: pl.BlockSpec((1, D), lambda s, b: (0, 0))
    in_specs = [
        pl.BlockSpec((blk, D), lambda s, b: (b * sb + s, 0)),
        pl.BlockSpec((blk, D), lambda s, b: (k_blk + s, 0)),
        vec(), vec(), vec(), vec(),
    ]
    args = [g, pos_table, tw.reshape(1, D), tb.reshape(1, D),
            pw.reshape(1, D), pb.reshape(1, D)]
    kwargs = {}
    if buf is not None:
        in_specs.append(pl.BlockSpec(memory_space=pl.ANY))
        args.append(buf)
        kwargs["input_output_aliases"] = {6: 0}
        body = lambda g_, p_, a_, b_, c_, d_, _buf, o_: _ln_body(
            g_, p_, a_, b_, c_, d_, o_)
    else:
        body = _ln_body
    return pl.pallas_call(
        body,
        grid=(sb, B),
        in_specs=in_specs,
        out_specs=pl.BlockSpec(
            (blk, D), lambda s, b: (b * (S // blk) + k_blk + s, 0)),
        out_shape=jax.ShapeDtypeStruct((N, D), jnp.float32),
        **kwargs,
    )(*args)


def kernel(input_ids, token_table, pos_table, tok_ln_w, tok_ln_b, pos_ln_w, pos_ln_b):
    B, S = input_ids.shape
    V, D = token_table.shape
    sizes = (1024, 1024, 1024, 1024)
    offs = (0, 1024, 2048, 3072)
    ids32 = input_ids.astype(jnp.int32)
    gs = [
        _sc_gather(ids32[:, o:o + sz].reshape(-1), token_table)
        for o, sz in zip(offs, sizes)
    ]
    buf = None
    for g, o, sz in zip(gs, offs, sizes):
        buf = _tc_ln_chunk(g, pos_table, tok_ln_w, tok_ln_b, pos_ln_w,
                           pos_ln_b, buf, o // 1024, B * S, B, S, sz)
    return buf.reshape(B, S, D)
